# baseline (speedup 1.0000x reference)
import functools

import jax
import jax.numpy as jnp
from jax import lax
from jax.experimental import pallas as pl
from jax.experimental.pallas import tpu as pltpu
from jax.experimental.pallas import tpu_sc as plsc

_VOCAB = 1000000
_DIM = 32
_BATCH = 4096
_SEQ = 50
_PAD_IDX = 0

_B = _BATCH * _SEQ
_NW = 32
_ROWS_W = _B // _NW
_CHUNK = 1600
_NCHUNK = _ROWS_W // _CHUNK
_LANES = 16


def _sc_embed(table, idx, pos):
  mesh = plsc.VectorSubcoreMesh(core_axis_name="c", subcore_axis_name="s")

  @functools.partial(
      pl.kernel,
      mesh=mesh,
      compiler_params=pltpu.CompilerParams(use_tc_tiling_on_sc=False),
      out_type=jax.ShapeDtypeStruct((_B, _DIM), jnp.float32),
      scratch_types=[
          pltpu.VMEM((_CHUNK,), jnp.int32),
          pltpu.VMEM((_CHUNK, _DIM), jnp.float32),
          pltpu.VMEM((_SEQ, _DIM), jnp.float32),
          pltpu.SemaphoreType.DMA,
      ],
  )
  def k(table_h, idx_h, pos_h, out_h, idx_v, rows_v, pos_v, sem):
    wid = lax.axis_index("s") * 2 + lax.axis_index("c")
    base = wid * _ROWS_W
    pltpu.sync_copy(pos_h, pos_v)
    lanes = lax.iota(jnp.int32, _LANES)

    def chunk_body(g, carry):
      cb = base + g * _CHUNK
      pltpu.sync_copy(idx_h.at[pl.ds(cb, _CHUNK)], idx_v)
      pltpu.async_copy(table_h.at[idx_v], rows_v, sem).wait()

      def grp_body(g, c2):
        rbase = g * _LANES
        keep = jnp.where(idx_v[pl.ds(rbase, _LANES)] != _PAD_IDX, 1.0, 0.0)
        for l in range(_LANES):
          r = rbase + l
          sr = lax.rem(r, _SEQ)
          s_l = keep[l]
          rows_v[r, pl.ds(0, 16)] = (
              rows_v[r, pl.ds(0, 16)] * s_l + pos_v[sr, pl.ds(0, 16)])
          rows_v[r, pl.ds(16, 16)] = (
              rows_v[r, pl.ds(16, 16)] * s_l + pos_v[sr, pl.ds(16, 16)])
        return c2

      lax.fori_loop(0, _CHUNK // _LANES, grp_body, 0)

      pltpu.sync_copy(rows_v, out_h.at[pl.ds(cb, _CHUNK)])
      return carry

    lax.fori_loop(0, _NCHUNK, chunk_body, 0)

  return k(table, idx, pos)


def kernel(x, embedding_table, pos_embeddings):
  idx = x.reshape(-1).astype(jnp.int32)
  out = _sc_embed(embedding_table, idx, pos_embeddings)
  return out.reshape(_BATCH, _SEQ, _DIM)
